# Initial kernel scaffold; baseline (speedup 1.0000x reference)
import functools

import jax
import jax.numpy as jnp
from jax import lax
from jax.experimental import pallas as pl
from jax.experimental.pallas import tpu as pltpu
from jax.experimental.pallas import tpu_sc as plsc

_NC = 2
_NS = 16
_LANES = 16


@functools.lru_cache(maxsize=None)
def _make_pool(B, L, V, DIM, G, BS):
  NW = _NC * _NS
  SPW = B // NW
  NBLK = SPW // BS
  GPB = BS * L // G
  GPS = L // G
  IDROWS_PW = SPW * L // G

  mesh = plsc.VectorSubcoreMesh(core_axis_name="c", subcore_axis_name="s")

  @functools.partial(
      pl.kernel,
      out_type=jax.ShapeDtypeStruct((B * DIM,), jnp.float32),
      mesh=mesh,
      scratch_types=[
          pltpu.VMEM((GPB, G), jnp.int32),
          pltpu.VMEM((2, G, DIM), jnp.float32),
          pltpu.VMEM((BS * DIM,), jnp.float32),
          pltpu.SemaphoreType.DMA,
          pltpu.SemaphoreType.DMA,
      ],
  )
  def pool(ids_hbm, emb_hbm, out_hbm, ids_v, rows_v, pooled_v, sem0, sem1):
    sems = (sem0, sem1)
    wid = lax.axis_index("s") * _NC + lax.axis_index("c")

    def block_body(blk, carry):
      idrow0 = wid * IDROWS_PW + blk * GPB
      pltpu.sync_copy(ids_hbm.at[pl.ds(idrow0, GPB), :], ids_v)

      copies = [None] * GPB
      copies[0] = pltpu.async_copy(
          emb_hbm.at[ids_v.at[0]], rows_v.at[0], sems[0])

      acc = None
      for j in range(GPB):
        if j + 1 < GPB:
          nb = (j + 1) % 2
          copies[j + 1] = pltpu.async_copy(
              emb_hbm.at[ids_v.at[j + 1]], rows_v.at[nb], sems[nb])
        copies[j].wait()

        s, h = divmod(j, GPS)
        if h == 0:
          acc = tuple(jnp.zeros((_LANES,), jnp.float32)
                      for _ in range(DIM // _LANES))
        buf = j % 2

        def acc_body(i, a, buf=buf):
          row = i * 4
          out = list(a)
          for u in range(4):
            for v in range(DIM // _LANES):
              out[v] = out[v] + rows_v[buf, row + u, pl.ds(v * _LANES, _LANES)]
          return tuple(out)

        acc = lax.fori_loop(0, G // 4, acc_body, acc)
        if h == GPS - 1:
          for v in range(DIM // _LANES):
            pooled_v[pl.ds(s * DIM + v * _LANES, _LANES)] = acc[v]

      pltpu.sync_copy(
          pooled_v,
          out_hbm.at[pl.ds(wid * SPW * DIM + blk * BS * DIM, BS * DIM)])
      return carry

    lax.fori_loop(0, NBLK, block_body, 0)

  return pool


def _linear(pooled, wt, b2):
  B, DIM = pooled.shape
  OUT = wt.shape[1]

  def body(x_ref, w_ref, b_ref, o_ref):
    o_ref[...] = jnp.dot(
        x_ref[...], w_ref[...], preferred_element_type=jnp.float32
    ) + b_ref[...]

  return pl.pallas_call(
      body,
      out_shape=jax.ShapeDtypeStruct((B, OUT), jnp.float32),
  )(pooled, wt, b2)


def kernel(ids, emb, W, b):
  B, L = ids.shape
  V, DIM = emb.shape
  OUT = W.shape[0]
  G = 100
  BS = 16

  ids2d = ids.reshape(B * L // G, G).astype(jnp.int32)
  sums = _make_pool(B, L, V, DIM, G, BS)(ids2d, emb)
  pooled = sums.reshape(B, DIM)
  wt = (W.astype(jnp.float32) * (1.0 / L)).T
  b2 = b.astype(jnp.float32).reshape(1, OUT)
  return _linear(pooled, wt, b2)


# speedup vs baseline: 11.3769x; 11.3769x over previous
import functools

import jax
import jax.numpy as jnp
from jax import lax
from jax.experimental import pallas as pl
from jax.experimental.pallas import tpu as pltpu
from jax.experimental.pallas import tpu_sc as plsc

_NC = 2
_NS = 16
_LANES = 16


@functools.lru_cache(maxsize=None)
def _make_pool(B, L, V, DIM, G, BS):
  NW = _NC * _NS
  SPW = B // NW
  NBLK = SPW // BS
  GPB = BS * L // G
  GPS = L // G
  IDROWS_PW = SPW * L // G

  mesh = plsc.VectorSubcoreMesh(core_axis_name="c", subcore_axis_name="s")

  @functools.partial(
      pl.kernel,
      out_type=jax.ShapeDtypeStruct((B * DIM,), jnp.float32),
      mesh=mesh,
      compiler_params=pltpu.CompilerParams(use_tc_tiling_on_sc=False),
      scratch_types=[
          pltpu.VMEM((GPB, G), jnp.int32),
          pltpu.VMEM((2, G, DIM), jnp.float32),
          pltpu.VMEM((BS * DIM,), jnp.float32),
          pltpu.SemaphoreType.DMA,
          pltpu.SemaphoreType.DMA,
      ],
  )
  def pool(ids_hbm, emb_hbm, out_hbm, ids_v, rows_v, pooled_v, sem0, sem1):
    sems = (sem0, sem1)
    wid = lax.axis_index("s") * _NC + lax.axis_index("c")

    def block_body(blk, carry):
      idrow0 = wid * IDROWS_PW + blk * GPB
      pltpu.sync_copy(ids_hbm.at[pl.ds(idrow0, GPB), :], ids_v)

      copies = [None] * GPB
      copies[0] = pltpu.async_copy(
          emb_hbm.at[ids_v.at[0]], rows_v.at[0], sems[0])

      acc = None
      for j in range(GPB):
        if j + 1 < GPB:
          nb = (j + 1) % 2
          copies[j + 1] = pltpu.async_copy(
              emb_hbm.at[ids_v.at[j + 1]], rows_v.at[nb], sems[nb])
        copies[j].wait()

        s, h = divmod(j, GPS)
        if h == 0:
          acc = tuple(jnp.zeros((_LANES,), jnp.float32)
                      for _ in range(DIM // _LANES))
        buf = j % 2

        def acc_body(i, a, buf=buf):
          row = i * 4
          out = list(a)
          for u in range(4):
            for v in range(DIM // _LANES):
              out[v] = out[v] + rows_v[buf, row + u, pl.ds(v * _LANES, _LANES)]
          return tuple(out)

        acc = lax.fori_loop(0, G // 4, acc_body, acc)
        if h == GPS - 1:
          for v in range(DIM // _LANES):
            pooled_v[pl.ds(s * DIM + v * _LANES, _LANES)] = acc[v]

      pltpu.sync_copy(
          pooled_v,
          out_hbm.at[pl.ds(wid * SPW * DIM + blk * BS * DIM, BS * DIM)])
      return carry

    lax.fori_loop(0, NBLK, block_body, 0)

  return pool


def _linear(pooled, wt, b2):
  B, DIM = pooled.shape
  OUT = wt.shape[1]

  def body(x_ref, w_ref, b_ref, o_ref):
    o_ref[...] = jnp.dot(
        x_ref[...], w_ref[...], preferred_element_type=jnp.float32
    ) + b_ref[...]

  return pl.pallas_call(
      body,
      out_shape=jax.ShapeDtypeStruct((B, OUT), jnp.float32),
  )(pooled, wt, b2)


def kernel(ids, emb, W, b):
  B, L = ids.shape
  V, DIM = emb.shape
  OUT = W.shape[0]
  G = 100
  BS = 16

  ids2d = ids.reshape(B * L // G, G).astype(jnp.int32)
  sums = _make_pool(B, L, V, DIM, G, BS)(ids2d, emb)
  pooled = sums.reshape(B, DIM)
  wt = (W.astype(jnp.float32) * (1.0 / L)).T
  b2 = b.astype(jnp.float32).reshape(1, OUT)
  return _linear(pooled, wt, b2)
